# Initial kernel scaffold; baseline (speedup 1.0000x reference)
import jax
import jax.numpy as jnp
from jax.experimental import pallas as pl


def kernel(x, edge_index, W1l, b1l, W1r, b1r, W2l, b2l, W2r, b2r, W3l, b3l, W3r, b3r):
    raise NotImplementedError("write your pallas kernel here")


# speedup vs baseline: 2.4571x; 2.4571x over previous
import jax
import jax.numpy as jnp
from jax import lax
from jax.experimental import pallas as pl
from jax.experimental.pallas import tpu as pltpu
from jax.experimental.pallas import tpu_sc as plsc

N = 10000
D = 128
NC = 2
NS = 16
NW = NC * NS
K = 128
G = 4
NPAD = 10112
RPT = NPAD // NS


def _chunks(total, step):
    out = []
    r = 0
    while r < total:
        n = min(step, total - r)
        out.append((r, n))
        r += n
    return out


def _make_agg(Q):
    mesh = plsc.VectorSubcoreMesh(core_axis_name="c", subcore_axis_name="s",
                                  num_cores=NC, num_subcores=NS)
    out_type = jax.ShapeDtypeStruct((NC, NPAD, D), jnp.float32)
    scratch = [
        pltpu.VMEM((2, G, K), jnp.int32),
        pltpu.VMEM((2, G, K), jnp.int32),
        pltpu.VMEM((K, D), jnp.float32),
        pltpu.VMEM((K, D), jnp.float32),
        pltpu.VMEM_SHARED((NPAD, D), jnp.float32),
        pltpu.SemaphoreType.DMA,
        pltpu.SemaphoreType.DMA,
        pltpu.SemaphoreType.DMA,
        pltpu.SemaphoreType.DMA,
    ]

    def body(h_hbm, src_hbm, dst_hbm, out_hbm, sring, dring, rows0, rows1,
             acc, sem0, sem1, semr0, semr1):
        c = lax.axis_index("c")
        s = lax.axis_index("s")
        wid = c * NS + s
        rows = (rows0, rows1)
        sems = (sem0, sem1)
        semr = (semr0, semr1)

        def zrow(i, _):
            rows0[i // (D // 16), pl.ds((i % (D // 16)) * 16, 16)] = (
                jnp.zeros((16,), jnp.float32))
            return 0
        lax.fori_loop(0, K * (D // 16), zrow, 0)
        r0 = s * RPT
        for r, n in _chunks(RPT, K):
            pltpu.sync_copy(rows0.at[pl.ds(0, n)], acc.at[pl.ds(r0 + r, n)])
        plsc.subcore_barrier()

        def start_ring(p, q):
            pltpu.async_copy(src_hbm.at[wid, q], sring.at[p], semr[p])
            pltpu.async_copy(dst_hbm.at[wid, q], dring.at[p], semr[p])

        def drain_ring(p):
            pltpu.make_async_copy(src_hbm.at[wid, 0], sring.at[p],
                                  semr[p]).wait()
            pltpu.make_async_copy(dst_hbm.at[wid, 0], dring.at[p],
                                  semr[p]).wait()

        def start_gather(p, j, rbuf, sem):
            pltpu.async_copy(h_hbm.at[sring.at[p, j]], rbuf, sem)

        def drain_gather(rbuf, sem):
            pltpu.make_async_copy(h_hbm.at[sring.at[0, 0]], rbuf, sem).wait()

        def half(i, p):
            if p == 0:
                start_ring(1, 2 * i + 1)
            else:
                start_ring(0, 2 * i + 2)
            for j in range(G):
                nxt = rows[(j + 1) % 2]
                if j < G - 1:
                    start_gather(p, j + 1, nxt, sems[(j + 1) % 2])
                else:
                    drain_ring(1 - p)
                    start_gather(1 - p, 0, nxt, sems[(j + 1) % 2])
                drain_gather(rows[j % 2], sems[j % 2])
                pltpu.sync_copy(rows[j % 2], acc.at[dring.at[p, j]],
                                add=True)

        pltpu.async_copy(src_hbm.at[wid, 0], sring.at[0], semr0)
        pltpu.async_copy(dst_hbm.at[wid, 0], dring.at[0], semr0)
        drain_ring(0)
        start_gather(0, 0, rows0, sem0)

        def step(i, _):
            half(i, 0)
            half(i, 1)
            return 0
        lax.fori_loop(0, Q // 2, step, 0)
        drain_gather(rows0, sem0)
        plsc.subcore_barrier()

        for r, n in _chunks(RPT, K):
            pltpu.sync_copy(acc.at[pl.ds(r0 + r, n)], rows0.at[pl.ds(0, n)])
            pltpu.sync_copy(rows0.at[pl.ds(0, n)],
                            out_hbm.at[c, pl.ds(r0 + r, n)])

    return pl.kernel(body, out_type=out_type, mesh=mesh,
                     scratch_types=scratch)


def _make_cnt(B):
    mesh = plsc.VectorSubcoreMesh(core_axis_name="c", subcore_axis_name="s",
                                  num_cores=NC, num_subcores=NS)
    out_type = jax.ShapeDtypeStruct((NC, NPAD, D), jnp.float32)
    scratch = [
        pltpu.VMEM((B, K), jnp.int32),
        pltpu.VMEM((K, D), jnp.float32),
        pltpu.VMEM_SHARED((NPAD, D), jnp.float32),
    ]

    def body(dst_hbm, cnt_hbm, dst_v, buf, cacc):
        c = lax.axis_index("c")
        s = lax.axis_index("s")
        wid = c * NS + s

        def fill(val):
            def row(i, _):
                buf[i // (D // 16), pl.ds((i % (D // 16)) * 16, 16)] = (
                    jnp.full((16,), val, jnp.float32))
                return 0
            lax.fori_loop(0, K * (D // 16), row, 0)

        fill(0.0)
        r0 = s * RPT
        for r, n in _chunks(RPT, K):
            pltpu.sync_copy(buf.at[pl.ds(0, n)], cacc.at[pl.ds(r0 + r, n)])
        fill(1.0)
        plsc.subcore_barrier()

        pltpu.sync_copy(dst_hbm.at[wid], dst_v)

        def step(b, _):
            pltpu.sync_copy(buf, cacc.at[dst_v.at[b]], add=True)
            return 0
        lax.fori_loop(0, B, step, 0)
        plsc.subcore_barrier()

        for r, n in _chunks(RPT, K):
            pltpu.sync_copy(cacc.at[pl.ds(r0 + r, n)], buf.at[pl.ds(0, n)])
            pltpu.sync_copy(buf.at[pl.ds(0, n)],
                            cnt_hbm.at[c, pl.ds(r0 + r, n)])

    return pl.kernel(body, out_type=out_type, mesh=mesh,
                     scratch_types=scratch)


def _tc_layer(h, a0, a1, c0, c1, wl, wr, b, relu):
    R = 1000

    def body(h_ref, a0_ref, a1_ref, c0_ref, c1_ref, wl_ref, wr_ref, b_ref,
             o_ref):
        cnt = c0_ref[:, 0:1] + c1_ref[:, 0:1]
        inv = 1.0 / jnp.maximum(cnt, 1.0)
        agg = (a0_ref[...] + a1_ref[...]) * inv
        y = jnp.dot(agg, wl_ref[...], preferred_element_type=jnp.float32)
        y = y + jnp.dot(h_ref[...], wr_ref[...],
                        preferred_element_type=jnp.float32)
        y = y + b_ref[...]
        if relu:
            y = jnp.maximum(y, 0.0)
        o_ref[...] = y

    row_spec = pl.BlockSpec((R, D), lambda i: (i, 0))
    cnt_spec = pl.BlockSpec((R, D), lambda i: (i, 0))
    full_spec = pl.BlockSpec((D, D), lambda i: (0, 0))
    bias_spec = pl.BlockSpec((1, D), lambda i: (0, 0))
    return pl.pallas_call(
        body,
        grid=(N // R,),
        in_specs=[row_spec, row_spec, row_spec, cnt_spec, cnt_spec,
                  full_spec, full_spec, bias_spec],
        out_specs=row_spec,
        out_shape=jax.ShapeDtypeStruct((N, D), jnp.float32),
    )(h, a0, a1, c0, c1, wl, wr, b)


def kernel(x, edge_index, W1l, b1l, W1r, b1r, W2l, b2l, W2r, b2r,
           W3l, b3l, W3r, b3r):
    E = edge_index.shape[1]
    Q = -(-E // (NW * G * K))
    if Q % 2:
        Q += 1
    src = edge_index[0]
    dst = edge_index[1]
    EP = NW * Q * G * K
    pad = EP - E
    src_p = jnp.concatenate([src, jnp.zeros((pad,), jnp.int32)])
    dst_p = jnp.concatenate([dst, jnp.full((pad,), N, jnp.int32)])
    src_p = src_p.reshape(NW, Q, G, K)
    dst_p = dst_p.reshape(NW, Q, G, K)
    src_p = jnp.concatenate(
        [src_p, jnp.zeros((NW, 1, G, K), jnp.int32)], axis=1)
    dst_p = jnp.concatenate(
        [dst_p, jnp.full((NW, 1, G, K), N, jnp.int32)], axis=1)

    agg_fn = _make_agg(Q)
    cnt_fn = _make_cnt(Q * G)

    cnts = cnt_fn(dst_p[:, :Q].reshape(NW, Q * G, K))
    parts1 = agg_fn(x, src_p, dst_p)
    c0 = cnts[0, :N]
    c1 = cnts[1, :N]
    h1 = _tc_layer(x, parts1[0, :N], parts1[1, :N], c0, c1,
                   W1l, W1r, (b1l + b1r).reshape(1, D), relu=True)
    parts2 = agg_fn(h1, src_p, dst_p)
    h2 = _tc_layer(h1, parts2[0, :N], parts2[1, :N], c0, c1,
                   W2l, W2r, (b2l + b2r).reshape(1, D), relu=True)
    parts3 = agg_fn(h2, src_p, dst_p)
    h3 = _tc_layer(h2, parts3[0, :N], parts3[1, :N], c0, c1,
                   W3l, W3r, (b3l + b3r).reshape(1, D), relu=False)
    return h3


# speedup vs baseline: 2.5664x; 1.0445x over previous
import jax
import jax.numpy as jnp
from jax import lax
from jax.experimental import pallas as pl
from jax.experimental.pallas import tpu as pltpu
from jax.experimental.pallas import tpu_sc as plsc

N = 10000
D = 128
NC = 2
NS = 16
NW = NC * NS
K = 128
G = 4
NPAD = 10112
RPT = NPAD // NS


def _chunks(total, step):
    out = []
    r = 0
    while r < total:
        n = min(step, total - r)
        out.append((r, n))
        r += n
    return out


def _make_agg(Q0, Q1):
    mesh = plsc.VectorSubcoreMesh(core_axis_name="c", subcore_axis_name="s",
                                  num_cores=NC, num_subcores=NS)
    out_type = jax.ShapeDtypeStruct((NC, NPAD, D), jnp.float32)
    scratch = [
        pltpu.VMEM((2, G, K), jnp.int32),
        pltpu.VMEM((2, G, K), jnp.int32),
        pltpu.VMEM((K, D), jnp.float32),
        pltpu.VMEM((K, D), jnp.float32),
        pltpu.VMEM_SHARED((NPAD, D), jnp.float32),
        pltpu.SemaphoreType.DMA,
        pltpu.SemaphoreType.DMA,
        pltpu.SemaphoreType.DMA,
        pltpu.SemaphoreType.DMA,
    ]

    def body(h_hbm, src_hbm, dst_hbm, out_hbm, sring, dring, rows0, rows1,
             acc, sem0, sem1, semr0, semr1):
        c = lax.axis_index("c")
        s = lax.axis_index("s")
        wid = c * NS + s
        rows = (rows0, rows1)
        sems = (sem0, sem1)
        semr = (semr0, semr1)

        def zrow(i, _):
            rows0[i // (D // 16), pl.ds((i % (D // 16)) * 16, 16)] = (
                jnp.zeros((16,), jnp.float32))
            return 0
        lax.fori_loop(0, K * (D // 16), zrow, 0)
        r0 = s * RPT
        for r, n in _chunks(RPT, K):
            pltpu.sync_copy(rows0.at[pl.ds(0, n)], acc.at[pl.ds(r0 + r, n)])
        plsc.subcore_barrier()

        def start_ring(p, q):
            pltpu.async_copy(src_hbm.at[wid, q], sring.at[p], semr[p])
            pltpu.async_copy(dst_hbm.at[wid, q], dring.at[p], semr[p])

        def drain_ring(p):
            pltpu.make_async_copy(src_hbm.at[wid, 0], sring.at[p],
                                  semr[p]).wait()
            pltpu.make_async_copy(dst_hbm.at[wid, 0], dring.at[p],
                                  semr[p]).wait()

        def start_gather(p, j, rbuf, sem):
            pltpu.async_copy(h_hbm.at[sring.at[p, j]], rbuf, sem)

        def drain_gather(rbuf, sem):
            pltpu.make_async_copy(h_hbm.at[sring.at[0, 0]], rbuf, sem).wait()

        def half(i, p):
            if p == 0:
                start_ring(1, 2 * i + 1)
            else:
                start_ring(0, 2 * i + 2)
            for j in range(G):
                nxt = rows[(j + 1) % 2]
                if j < G - 1:
                    start_gather(p, j + 1, nxt, sems[(j + 1) % 2])
                else:
                    drain_ring(1 - p)
                    start_gather(1 - p, 0, nxt, sems[(j + 1) % 2])
                drain_gather(rows[j % 2], sems[j % 2])
                pltpu.sync_copy(rows[j % 2], acc.at[dring.at[p, j]],
                                add=True)

        pltpu.async_copy(src_hbm.at[wid, 0], sring.at[0], semr0)
        pltpu.async_copy(dst_hbm.at[wid, 0], dring.at[0], semr0)
        drain_ring(0)
        start_gather(0, 0, rows0, sem0)

        def step(i, _):
            half(i, 0)
            half(i, 1)
            return 0
        qc2 = jnp.where(c == 0, Q0 // 2, Q1 // 2)
        lax.fori_loop(0, qc2, step, 0)
        drain_gather(rows0, sem0)
        plsc.subcore_barrier()

        for r, n in _chunks(RPT, K):
            pltpu.sync_copy(acc.at[pl.ds(r0 + r, n)], rows0.at[pl.ds(0, n)])
            pltpu.sync_copy(rows0.at[pl.ds(0, n)],
                            out_hbm.at[c, pl.ds(r0 + r, n)])

    return pl.kernel(body, out_type=out_type, mesh=mesh,
                     scratch_types=scratch)


def _make_cnt(B):
    mesh = plsc.VectorSubcoreMesh(core_axis_name="c", subcore_axis_name="s",
                                  num_cores=NC, num_subcores=NS)
    out_type = jax.ShapeDtypeStruct((NC, NPAD, D), jnp.float32)
    scratch = [
        pltpu.VMEM((B, K), jnp.int32),
        pltpu.VMEM((K, D), jnp.float32),
        pltpu.VMEM_SHARED((NPAD, D), jnp.float32),
    ]

    def body(dst_hbm, cnt_hbm, dst_v, buf, cacc):
        c = lax.axis_index("c")
        s = lax.axis_index("s")
        wid = c * NS + s

        def fill(val):
            def row(i, _):
                buf[i // (D // 16), pl.ds((i % (D // 16)) * 16, 16)] = (
                    jnp.full((16,), val, jnp.float32))
                return 0
            lax.fori_loop(0, K * (D // 16), row, 0)

        fill(0.0)
        r0 = s * RPT
        for r, n in _chunks(RPT, K):
            pltpu.sync_copy(buf.at[pl.ds(0, n)], cacc.at[pl.ds(r0 + r, n)])
        fill(1.0)
        plsc.subcore_barrier()

        pltpu.sync_copy(dst_hbm.at[wid], dst_v)

        def step(b, _):
            pltpu.sync_copy(buf, cacc.at[dst_v.at[b]], add=True)
            return 0
        lax.fori_loop(0, B, step, 0)
        plsc.subcore_barrier()

        for r, n in _chunks(RPT, K):
            pltpu.sync_copy(cacc.at[pl.ds(r0 + r, n)], buf.at[pl.ds(0, n)])
            pltpu.sync_copy(buf.at[pl.ds(0, n)],
                            cnt_hbm.at[c, pl.ds(r0 + r, n)])

    return pl.kernel(body, out_type=out_type, mesh=mesh,
                     scratch_types=scratch)


_ROW_SPEC = pl.BlockSpec((1000, D), lambda i: (i, 0))
_FULL_SPEC = pl.BlockSpec((D, D), lambda i: (0, 0))
_BIAS_SPEC = pl.BlockSpec((1, D), lambda i: (0, 0))


def _tc_right(h, wr, b):
    def body(h_ref, wr_ref, b_ref, o_ref):
        o_ref[...] = jnp.dot(h_ref[...], wr_ref[...],
                             preferred_element_type=jnp.float32) + b_ref[...]
    return pl.pallas_call(
        body,
        grid=(N // 1000,),
        in_specs=[_ROW_SPEC, _FULL_SPEC, _BIAS_SPEC],
        out_specs=_ROW_SPEC,
        out_shape=jax.ShapeDtypeStruct((N, D), jnp.float32),
    )(h, wr, b)


def _tc_left(a0, a1, c0, c1, wl, ha, relu):
    def body(a0_ref, a1_ref, c0_ref, c1_ref, wl_ref, ha_ref, o_ref):
        cnt = c0_ref[:, 0:1] + c1_ref[:, 0:1]
        inv = 1.0 / jnp.maximum(cnt, 1.0)
        agg = (a0_ref[...] + a1_ref[...]) * inv
        y = jnp.dot(agg, wl_ref[...], preferred_element_type=jnp.float32)
        y = y + ha_ref[...]
        if relu:
            y = jnp.maximum(y, 0.0)
        o_ref[...] = y

    return pl.pallas_call(
        body,
        grid=(N // 1000,),
        in_specs=[_ROW_SPEC, _ROW_SPEC, _ROW_SPEC, _ROW_SPEC,
                  _FULL_SPEC, _ROW_SPEC],
        out_specs=_ROW_SPEC,
        out_shape=jax.ShapeDtypeStruct((N, D), jnp.float32),
    )(a0, a1, c0, c1, wl, ha)


def _pad_half(arr, fill, Qc, Qmax):
    cap = NS * Qc * G * K
    pad = cap - arr.shape[0]
    a = jnp.concatenate([arr, jnp.full((pad,), fill, jnp.int32)])
    a = a.reshape(NS, Qc, G, K)
    tail = jnp.full((NS, Qmax + 1 - Qc, G, K), fill, jnp.int32)
    return jnp.concatenate([a, tail], axis=1)


def kernel(x, edge_index, W1l, b1l, W1r, b1r, W2l, b2l, W2r, b2r,
           W3l, b3l, W3r, b3r):
    E = edge_index.shape[1]
    SB = NS * G * K
    src = edge_index[0]
    dst = edge_index[1]

    Q0 = max(2, int(round(0.70 * E / SB / 2)) * 2)
    E0 = min(NS * Q0 * G * K, E)
    Q0 = -(-E0 // SB)
    Q0 += Q0 % 2
    Q1 = max(2, -(-(E - E0) // SB))
    Q1 += Q1 % 2
    Qmax = max(Q0, Q1)
    src_p = jnp.concatenate([_pad_half(src[:E0], 0, Q0, Qmax),
                             _pad_half(src[E0:], 0, Q1, Qmax)], axis=0)
    dst_p = jnp.concatenate([_pad_half(dst[:E0], N, Q0, Qmax),
                             _pad_half(dst[E0:], N, Q1, Qmax)], axis=0)

    Qb = -(-E // (NW * G * K))
    EPb = NW * Qb * G * K
    dst_b = jnp.concatenate([dst, jnp.full((EPb - E,), N, jnp.int32)])
    dst_b = dst_b.reshape(NW, Qb * G, K)

    agg_fn = _make_agg(Q0, Q1)
    cnt_fn = _make_cnt(Qb * G)

    cnts = cnt_fn(dst_b)
    c0 = cnts[0, :N]
    c1 = cnts[1, :N]

    h = x
    for wl, bl, wr, br, relu in ((W1l, b1l, W1r, b1r, True),
                                 (W2l, b2l, W2r, b2r, True),
                                 (W3l, b3l, W3r, b3r, False)):
        ha = _tc_right(h, wr, (bl + br).reshape(1, D))
        parts = agg_fn(h, src_p, dst_p)
        h = _tc_left(parts[0, :N], parts[1, :N], c0, c1, wl, ha, relu)
    return h


# speedup vs baseline: 2.6964x; 1.0507x over previous
import jax
import jax.numpy as jnp
from jax import lax
from jax.experimental import pallas as pl
from jax.experimental.pallas import tpu as pltpu
from jax.experimental.pallas import tpu_sc as plsc

N = 10000
D = 128
NC = 2
NS = 16
NW = NC * NS
K = 128
G = 4
NPAD = 10112
RPT = NPAD // NS


def _chunks(total, step):
    out = []
    r = 0
    while r < total:
        n = min(step, total - r)
        out.append((r, n))
        r += n
    return out


def _make_agg(Q0, Q1):
    mesh = plsc.VectorSubcoreMesh(core_axis_name="c", subcore_axis_name="s",
                                  num_cores=NC, num_subcores=NS)
    out_type = jax.ShapeDtypeStruct((NC, NPAD, D), jnp.float32)
    scratch = [
        pltpu.VMEM((2, G, K), jnp.int32),
        pltpu.VMEM((2, G, K), jnp.int32),
        pltpu.VMEM((K, D), jnp.float32),
        pltpu.VMEM((K, D), jnp.float32),
        pltpu.VMEM_SHARED((NPAD, D), jnp.float32),
        pltpu.SemaphoreType.DMA,
        pltpu.SemaphoreType.DMA,
        pltpu.SemaphoreType.DMA,
        pltpu.SemaphoreType.DMA,
    ]

    def body(h_hbm, src_hbm, dst_hbm, out_hbm, sring, dring, rows0, rows1,
             acc, sem0, sem1, semr0, semr1):
        c = lax.axis_index("c")
        s = lax.axis_index("s")
        wid = c * NS + s
        rows = (rows0, rows1)
        sems = (sem0, sem1)
        semr = (semr0, semr1)

        def zrow(i, _):
            rows0[i // (D // 16), pl.ds((i % (D // 16)) * 16, 16)] = (
                jnp.zeros((16,), jnp.float32))
            return 0
        lax.fori_loop(0, K * (D // 16), zrow, 0)
        r0 = s * RPT
        for r, n in _chunks(RPT, K):
            pltpu.sync_copy(rows0.at[pl.ds(0, n)], acc.at[pl.ds(r0 + r, n)])
        plsc.subcore_barrier()

        def start_ring(p, q):
            pltpu.async_copy(src_hbm.at[wid, q], sring.at[p], semr[p])
            pltpu.async_copy(dst_hbm.at[wid, q], dring.at[p], semr[p])

        def drain_ring(p):
            pltpu.make_async_copy(src_hbm.at[wid, 0], sring.at[p],
                                  semr[p]).wait()
            pltpu.make_async_copy(dst_hbm.at[wid, 0], dring.at[p],
                                  semr[p]).wait()

        def start_gather(p, j, rbuf, sem):
            pltpu.async_copy(h_hbm.at[sring.at[p, j]], rbuf, sem)

        def drain_gather(rbuf, sem):
            pltpu.make_async_copy(h_hbm.at[sring.at[0, 0]], rbuf, sem).wait()

        def half(i, p):
            if p == 0:
                start_ring(1, 2 * i + 1)
            else:
                start_ring(0, 2 * i + 2)
            for j in range(G):
                nxt = rows[(j + 1) % 2]
                if j < G - 1:
                    start_gather(p, j + 1, nxt, sems[(j + 1) % 2])
                else:
                    drain_ring(1 - p)
                    start_gather(1 - p, 0, nxt, sems[(j + 1) % 2])
                drain_gather(rows[j % 2], sems[j % 2])
                pltpu.sync_copy(rows[j % 2], acc.at[dring.at[p, j]],
                                add=True)

        pltpu.async_copy(src_hbm.at[wid, 0], sring.at[0], semr0)
        pltpu.async_copy(dst_hbm.at[wid, 0], dring.at[0], semr0)
        drain_ring(0)
        start_gather(0, 0, rows0, sem0)

        def step(i, _):
            half(i, 0)
            half(i, 1)
            return 0
        qc2 = jnp.where(c == 0, Q0 // 2, Q1 // 2)
        lax.fori_loop(0, qc2, step, 0)
        drain_gather(rows0, sem0)
        plsc.subcore_barrier()

        for r, n in _chunks(RPT, K):
            pltpu.sync_copy(acc.at[pl.ds(r0 + r, n)], rows0.at[pl.ds(0, n)])
            pltpu.sync_copy(rows0.at[pl.ds(0, n)],
                            out_hbm.at[c, pl.ds(r0 + r, n)])

    return pl.kernel(body, out_type=out_type, mesh=mesh,
                     scratch_types=scratch)


def _make_cnt(B):
    mesh = plsc.VectorSubcoreMesh(core_axis_name="c", subcore_axis_name="s",
                                  num_cores=NC, num_subcores=NS)
    out_type = jax.ShapeDtypeStruct((NC, NPAD, D), jnp.float32)
    scratch = [
        pltpu.VMEM((B, K), jnp.int32),
        pltpu.VMEM((K, D), jnp.float32),
        pltpu.VMEM_SHARED((NPAD, D), jnp.float32),
    ]

    def body(dst_hbm, cnt_hbm, dst_v, buf, cacc):
        c = lax.axis_index("c")
        s = lax.axis_index("s")
        wid = c * NS + s

        def fill(val):
            def row(i, _):
                buf[i // (D // 16), pl.ds((i % (D // 16)) * 16, 16)] = (
                    jnp.full((16,), val, jnp.float32))
                return 0
            lax.fori_loop(0, K * (D // 16), row, 0)

        fill(0.0)
        r0 = s * RPT
        for r, n in _chunks(RPT, K):
            pltpu.sync_copy(buf.at[pl.ds(0, n)], cacc.at[pl.ds(r0 + r, n)])
        fill(1.0)
        plsc.subcore_barrier()

        pltpu.sync_copy(dst_hbm.at[wid], dst_v)

        def step(b, _):
            pltpu.sync_copy(buf, cacc.at[dst_v.at[b]], add=True)
            return 0
        lax.fori_loop(0, B, step, 0)
        plsc.subcore_barrier()

        for r, n in _chunks(RPT, K):
            pltpu.sync_copy(cacc.at[pl.ds(r0 + r, n)], buf.at[pl.ds(0, n)])
            pltpu.sync_copy(buf.at[pl.ds(0, n)],
                            cnt_hbm.at[c, pl.ds(r0 + r, n)])

    return pl.kernel(body, out_type=out_type, mesh=mesh,
                     scratch_types=scratch)


_ROW_SPEC = pl.BlockSpec((1000, D), lambda i: (i, 0))
_FULL_SPEC = pl.BlockSpec((D, D), lambda i: (0, 0))
_BIAS_SPEC = pl.BlockSpec((1, D), lambda i: (0, 0))


def _tc_right(h, wr, b):
    def body(h_ref, wr_ref, b_ref, o_ref):
        o_ref[...] = jnp.dot(h_ref[...], wr_ref[...],
                             preferred_element_type=jnp.float32) + b_ref[...]
    return pl.pallas_call(
        body,
        grid=(N // 1000,),
        in_specs=[_ROW_SPEC, _FULL_SPEC, _BIAS_SPEC],
        out_specs=_ROW_SPEC,
        out_shape=jax.ShapeDtypeStruct((N, D), jnp.float32),
    )(h, wr, b)


def _tc_left(a0, a1, c0, c1, wl, ha, relu):
    def body(a0_ref, a1_ref, c0_ref, c1_ref, wl_ref, ha_ref, o_ref):
        cnt = c0_ref[:, 0:1] + c1_ref[:, 0:1]
        inv = 1.0 / jnp.maximum(cnt, 1.0)
        agg = (a0_ref[...] + a1_ref[...]) * inv
        y = jnp.dot(agg, wl_ref[...], preferred_element_type=jnp.float32)
        y = y + ha_ref[...]
        if relu:
            y = jnp.maximum(y, 0.0)
        o_ref[...] = y

    return pl.pallas_call(
        body,
        grid=(N // 1000,),
        in_specs=[_ROW_SPEC, _ROW_SPEC, _ROW_SPEC, _ROW_SPEC,
                  _FULL_SPEC, _ROW_SPEC],
        out_specs=_ROW_SPEC,
        out_shape=jax.ShapeDtypeStruct((N, D), jnp.float32),
    )(a0, a1, c0, c1, wl, ha)


def _pad_half(arr, fill, Qc, Qmax):
    cap = NS * Qc * G * K
    pad = cap - arr.shape[0]
    a = jnp.concatenate([arr, jnp.full((pad,), fill, jnp.int32)])
    a = a.reshape(NS, Qc, G, K)
    tail = jnp.full((NS, Qmax + 1 - Qc, G, K), fill, jnp.int32)
    return jnp.concatenate([a, tail], axis=1)


def kernel(x, edge_index, W1l, b1l, W1r, b1r, W2l, b2l, W2r, b2r,
           W3l, b3l, W3r, b3r):
    E = edge_index.shape[1]
    SB = NS * G * K
    src = edge_index[0]
    dst = edge_index[1]

    Q0 = max(2, int(round(0.85 * E / SB / 2)) * 2)
    E0 = min(NS * Q0 * G * K, E)
    Q0 = -(-E0 // SB)
    Q0 += Q0 % 2
    Q1 = max(2, -(-(E - E0) // SB))
    Q1 += Q1 % 2
    Qmax = max(Q0, Q1)
    src_p = jnp.concatenate([_pad_half(src[:E0], 0, Q0, Qmax),
                             _pad_half(src[E0:], 0, Q1, Qmax)], axis=0)
    dst_p = jnp.concatenate([_pad_half(dst[:E0], N, Q0, Qmax),
                             _pad_half(dst[E0:], N, Q1, Qmax)], axis=0)

    Qb = -(-E // (NW * G * K))
    EPb = NW * Qb * G * K
    dst_b = jnp.concatenate([dst, jnp.full((EPb - E,), N, jnp.int32)])
    dst_b = dst_b.reshape(NW, Qb * G, K)

    agg_fn = _make_agg(Q0, Q1)
    cnt_fn = _make_cnt(Qb * G)

    cnts = cnt_fn(dst_b)
    c0 = cnts[0, :N]
    c1 = cnts[1, :N]

    h = x
    for wl, bl, wr, br, relu in ((W1l, b1l, W1r, b1r, True),
                                 (W2l, b2l, W2r, b2r, True),
                                 (W3l, b3l, W3r, b3r, False)):
        ha = _tc_right(h, wr, (bl + br).reshape(1, D))
        parts = agg_fn(h, src_p, dst_p)
        h = _tc_left(parts[0, :N], parts[1, :N], c0, c1, wl, ha, relu)
    return h


# speedup vs baseline: 2.7214x; 1.0093x over previous
import jax
import jax.numpy as jnp
from jax import lax
from jax.experimental import pallas as pl
from jax.experimental.pallas import tpu as pltpu
from jax.experimental.pallas import tpu_sc as plsc

N = 10000
D = 128
NC = 2
NS = 16
NW = NC * NS
K = 128
G = 4
NPAD = 10112
RPT = NPAD // NS


def _chunks(total, step):
    out = []
    r = 0
    while r < total:
        n = min(step, total - r)
        out.append((r, n))
        r += n
    return out


def _make_agg(Q0, Q1):
    mesh = plsc.VectorSubcoreMesh(core_axis_name="c", subcore_axis_name="s",
                                  num_cores=NC, num_subcores=NS)
    out_type = jax.ShapeDtypeStruct((NC, NPAD, D), jnp.float32)
    scratch = [
        pltpu.VMEM((2, G, K), jnp.int32),
        pltpu.VMEM((2, G, K), jnp.int32),
        pltpu.VMEM((K, D), jnp.float32),
        pltpu.VMEM((K, D), jnp.float32),
        pltpu.VMEM_SHARED((NPAD, D), jnp.float32),
        pltpu.SemaphoreType.DMA,
        pltpu.SemaphoreType.DMA,
        pltpu.SemaphoreType.DMA,
        pltpu.SemaphoreType.DMA,
    ]

    def body(h_hbm, src_hbm, dst_hbm, out_hbm, sring, dring, rows0, rows1,
             acc, sem0, sem1, semr0, semr1):
        c = lax.axis_index("c")
        s = lax.axis_index("s")
        wid = c * NS + s
        rows = (rows0, rows1)
        sems = (sem0, sem1)
        semr = (semr0, semr1)

        def zrow(i, _):
            rows0[i // (D // 16), pl.ds((i % (D // 16)) * 16, 16)] = (
                jnp.zeros((16,), jnp.float32))
            return 0
        lax.fori_loop(0, K * (D // 16), zrow, 0)
        r0 = s * RPT
        for r, n in _chunks(RPT, K):
            pltpu.sync_copy(rows0.at[pl.ds(0, n)], acc.at[pl.ds(r0 + r, n)])
        plsc.subcore_barrier()

        def start_ring(p, q):
            pltpu.async_copy(src_hbm.at[wid, q], sring.at[p], semr[p])
            pltpu.async_copy(dst_hbm.at[wid, q], dring.at[p], semr[p])

        def drain_ring(p):
            pltpu.make_async_copy(src_hbm.at[wid, 0], sring.at[p],
                                  semr[p]).wait()
            pltpu.make_async_copy(dst_hbm.at[wid, 0], dring.at[p],
                                  semr[p]).wait()

        def start_gather(p, j, rbuf, sem):
            pltpu.async_copy(h_hbm.at[sring.at[p, j]], rbuf, sem)

        def drain_gather(rbuf, sem):
            pltpu.make_async_copy(h_hbm.at[sring.at[0, 0]], rbuf, sem).wait()

        def half(i, p):
            if p == 0:
                start_ring(1, 2 * i + 1)
            else:
                start_ring(0, 2 * i + 2)
            for j in range(G):
                nxt = rows[(j + 1) % 2]
                if j < G - 1:
                    start_gather(p, j + 1, nxt, sems[(j + 1) % 2])
                else:
                    drain_ring(1 - p)
                    start_gather(1 - p, 0, nxt, sems[(j + 1) % 2])
                drain_gather(rows[j % 2], sems[j % 2])
                pltpu.sync_copy(rows[j % 2], acc.at[dring.at[p, j]],
                                add=True)

        def step(i, _):
            half(i, 0)
            half(i, 1)
            return 0

        def pipeline(qc2):
            pltpu.async_copy(src_hbm.at[wid, 0], sring.at[0], semr0)
            pltpu.async_copy(dst_hbm.at[wid, 0], dring.at[0], semr0)
            drain_ring(0)
            start_gather(0, 0, rows0, sem0)
            lax.fori_loop(0, qc2, step, 0)
            drain_gather(rows0, sem0)

        if Q1 == 0:
            @pl.when(c == 0)
            def _():
                pipeline(Q0 // 2)
        else:
            pipeline(jnp.where(c == 0, Q0 // 2, Q1 // 2))
        plsc.subcore_barrier()

        for r, n in _chunks(RPT, K):
            pltpu.sync_copy(acc.at[pl.ds(r0 + r, n)], rows0.at[pl.ds(0, n)])
            pltpu.sync_copy(rows0.at[pl.ds(0, n)],
                            out_hbm.at[c, pl.ds(r0 + r, n)])

    return pl.kernel(body, out_type=out_type, mesh=mesh,
                     scratch_types=scratch)


def _make_cnt(B):
    mesh = plsc.VectorSubcoreMesh(core_axis_name="c", subcore_axis_name="s",
                                  num_cores=NC, num_subcores=NS)
    out_type = jax.ShapeDtypeStruct((NC, NPAD, D), jnp.float32)
    scratch = [
        pltpu.VMEM((B, K), jnp.int32),
        pltpu.VMEM((K, D), jnp.float32),
        pltpu.VMEM_SHARED((NPAD, D), jnp.float32),
    ]

    def body(dst_hbm, cnt_hbm, dst_v, buf, cacc):
        c = lax.axis_index("c")
        s = lax.axis_index("s")
        wid = c * NS + s

        def fill(val):
            def row(i, _):
                buf[i // (D // 16), pl.ds((i % (D // 16)) * 16, 16)] = (
                    jnp.full((16,), val, jnp.float32))
                return 0
            lax.fori_loop(0, K * (D // 16), row, 0)

        fill(0.0)
        r0 = s * RPT
        for r, n in _chunks(RPT, K):
            pltpu.sync_copy(buf.at[pl.ds(0, n)], cacc.at[pl.ds(r0 + r, n)])
        fill(1.0)
        plsc.subcore_barrier()

        pltpu.sync_copy(dst_hbm.at[wid], dst_v)

        def step(b, _):
            pltpu.sync_copy(buf, cacc.at[dst_v.at[b]], add=True)
            return 0
        lax.fori_loop(0, B, step, 0)
        plsc.subcore_barrier()

        for r, n in _chunks(RPT, K):
            pltpu.sync_copy(cacc.at[pl.ds(r0 + r, n)], buf.at[pl.ds(0, n)])
            pltpu.sync_copy(buf.at[pl.ds(0, n)],
                            cnt_hbm.at[c, pl.ds(r0 + r, n)])

    return pl.kernel(body, out_type=out_type, mesh=mesh,
                     scratch_types=scratch)


_ROW_SPEC = pl.BlockSpec((1000, D), lambda i: (i, 0))
_FULL_SPEC = pl.BlockSpec((D, D), lambda i: (0, 0))
_BIAS_SPEC = pl.BlockSpec((1, D), lambda i: (0, 0))


def _tc_right(h, wr, b):
    def body(h_ref, wr_ref, b_ref, o_ref):
        o_ref[...] = jnp.dot(h_ref[...], wr_ref[...],
                             preferred_element_type=jnp.float32) + b_ref[...]
    return pl.pallas_call(
        body,
        grid=(N // 1000,),
        in_specs=[_ROW_SPEC, _FULL_SPEC, _BIAS_SPEC],
        out_specs=_ROW_SPEC,
        out_shape=jax.ShapeDtypeStruct((N, D), jnp.float32),
    )(h, wr, b)


def _tc_left(a0, a1, c0, c1, wl, ha, relu):
    def body(a0_ref, a1_ref, c0_ref, c1_ref, wl_ref, ha_ref, o_ref):
        cnt = c0_ref[:, 0:1] + c1_ref[:, 0:1]
        inv = 1.0 / jnp.maximum(cnt, 1.0)
        agg = (a0_ref[...] + a1_ref[...]) * inv
        y = jnp.dot(agg, wl_ref[...], preferred_element_type=jnp.float32)
        y = y + ha_ref[...]
        if relu:
            y = jnp.maximum(y, 0.0)
        o_ref[...] = y

    return pl.pallas_call(
        body,
        grid=(N // 1000,),
        in_specs=[_ROW_SPEC, _ROW_SPEC, _ROW_SPEC, _ROW_SPEC,
                  _FULL_SPEC, _ROW_SPEC],
        out_specs=_ROW_SPEC,
        out_shape=jax.ShapeDtypeStruct((N, D), jnp.float32),
    )(a0, a1, c0, c1, wl, ha)


def _pad_half(arr, fill, Qc, Qmax):
    cap = NS * Qc * G * K
    pad = cap - arr.shape[0]
    a = jnp.concatenate([arr, jnp.full((pad,), fill, jnp.int32)])
    a = a.reshape(NS, Qc, G, K)
    tail = jnp.full((NS, Qmax + 1 - Qc, G, K), fill, jnp.int32)
    return jnp.concatenate([a, tail], axis=1)


def kernel(x, edge_index, W1l, b1l, W1r, b1r, W2l, b2l, W2r, b2r,
           W3l, b3l, W3r, b3r):
    E = edge_index.shape[1]
    SB = NS * G * K
    src = edge_index[0]
    dst = edge_index[1]

    Q0 = -(-E // SB)
    Q0 += Q0 % 2
    E0 = E
    Q1 = 0
    Qmax = Q0
    src_p = jnp.concatenate([_pad_half(src[:E0], 0, Q0, Qmax),
                             _pad_half(src[E0:], 0, Q1, Qmax)], axis=0)
    dst_p = jnp.concatenate([_pad_half(dst[:E0], N, Q0, Qmax),
                             _pad_half(dst[E0:], N, Q1, Qmax)], axis=0)

    Qb = -(-E // (NW * G * K))
    EPb = NW * Qb * G * K
    dst_b = jnp.concatenate([dst, jnp.full((EPb - E,), N, jnp.int32)])
    dst_b = dst_b.reshape(NW, Qb * G, K)

    agg_fn = _make_agg(Q0, Q1)
    cnt_fn = _make_cnt(Qb * G)

    cnts = cnt_fn(dst_b)
    c0 = cnts[0, :N]
    c1 = cnts[1, :N]

    h = x
    for wl, bl, wr, br, relu in ((W1l, b1l, W1r, b1r, True),
                                 (W2l, b2l, W2r, b2r, True),
                                 (W3l, b3l, W3r, b3r, False)):
        ha = _tc_right(h, wr, (bl + br).reshape(1, D))
        parts = agg_fn(h, src_p, dst_p)
        h = _tc_left(parts[0, :N], parts[1, :N], c0, c1, wl, ha, relu)
    return h


# speedup vs baseline: 3.1577x; 1.1603x over previous
import jax
import jax.numpy as jnp
from jax import lax
from jax.experimental import pallas as pl
from jax.experimental.pallas import tpu as pltpu
from jax.experimental.pallas import tpu_sc as plsc

N = 10000
D = 128
NC = 2
NS = 16
NW = NC * NS
K = 128
G = 4
NPAD = 10112
RPT = NPAD // NS


def _chunks(total, step):
    out = []
    r = 0
    while r < total:
        n = min(step, total - r)
        out.append((r, n))
        r += n
    return out


def _make_agg(Q0, Q1, CH=0, CB=0):
    with_cnt = CH > 0
    mesh = plsc.VectorSubcoreMesh(core_axis_name="c", subcore_axis_name="s",
                                  num_cores=NC, num_subcores=NS)
    out_type = jax.ShapeDtypeStruct((NC, NPAD, D), jnp.float32)
    scratch = [
        pltpu.VMEM((2, G, K), jnp.int32),
        pltpu.VMEM((2, G, K), jnp.int32),
        pltpu.VMEM((K, D), jnp.float32),
        pltpu.VMEM((K, D), jnp.float32),
        pltpu.VMEM_SHARED((NPAD, D), jnp.float32),
        pltpu.SemaphoreType.DMA,
        pltpu.SemaphoreType.DMA,
        pltpu.SemaphoreType.DMA,
        pltpu.SemaphoreType.DMA,
    ]
    if with_cnt:
        scratch.append(pltpu.VMEM((CB, K), jnp.int32))

    def body(h_hbm, src_hbm, dst_hbm, *refs):
        if with_cnt:
            (dstc_hbm, out_hbm, sring, dring, rows0, rows1, acc,
             sem0, sem1, semr0, semr1, dstv) = refs
        else:
            (out_hbm, sring, dring, rows0, rows1, acc,
             sem0, sem1, semr0, semr1) = refs
        c = lax.axis_index("c")
        s = lax.axis_index("s")
        wid = c * NS + s
        rows = (rows0, rows1)
        sems = (sem0, sem1)
        semr = (semr0, semr1)

        def zrow(i, _):
            rows0[i // (D // 16), pl.ds((i % (D // 16)) * 16, 16)] = (
                jnp.zeros((16,), jnp.float32))
            return 0
        lax.fori_loop(0, K * (D // 16), zrow, 0)
        r0 = s * RPT
        for r, n in _chunks(RPT, K):
            pltpu.sync_copy(rows0.at[pl.ds(0, n)], acc.at[pl.ds(r0 + r, n)])
        plsc.subcore_barrier()

        def start_ring(p, q):
            pltpu.async_copy(src_hbm.at[wid, q], sring.at[p], semr[p])
            pltpu.async_copy(dst_hbm.at[wid, q], dring.at[p], semr[p])

        def drain_ring(p):
            pltpu.make_async_copy(src_hbm.at[wid, 0], sring.at[p],
                                  semr[p]).wait()
            pltpu.make_async_copy(dst_hbm.at[wid, 0], dring.at[p],
                                  semr[p]).wait()

        def start_gather(p, j, rbuf, sem):
            pltpu.async_copy(h_hbm.at[sring.at[p, j]], rbuf, sem)

        def drain_gather(rbuf, sem):
            pltpu.make_async_copy(h_hbm.at[sring.at[0, 0]], rbuf, sem).wait()

        def half(i, p):
            if p == 0:
                start_ring(1, 2 * i + 1)
            else:
                start_ring(0, 2 * i + 2)
            for j in range(G):
                nxt = rows[(j + 1) % 2]
                if j < G - 1:
                    start_gather(p, j + 1, nxt, sems[(j + 1) % 2])
                else:
                    drain_ring(1 - p)
                    start_gather(1 - p, 0, nxt, sems[(j + 1) % 2])
                drain_gather(rows[j % 2], sems[j % 2])
                pltpu.sync_copy(rows[j % 2], acc.at[dring.at[p, j]],
                                add=True)

        def step(i, _):
            half(i, 0)
            half(i, 1)
            return 0

        def pipeline(qc2):
            pltpu.async_copy(src_hbm.at[wid, 0], sring.at[0], semr0)
            pltpu.async_copy(dst_hbm.at[wid, 0], dring.at[0], semr0)
            drain_ring(0)
            start_gather(0, 0, rows0, sem0)
            lax.fori_loop(0, qc2, step, 0)
            drain_gather(rows0, sem0)

        if Q1 == 0:
            @pl.when(c == 0)
            def _():
                pipeline(Q0 // 2)
        else:
            pipeline(jnp.where(c == 0, Q0 // 2, Q1 // 2))

        if with_cnt:
            @pl.when(c == 1)
            def _():
                def onerow(i, _):
                    rows0[i // (D // 16), pl.ds((i % (D // 16)) * 16, 16)] = (
                        jnp.ones((16,), jnp.float32))
                    return 0
                lax.fori_loop(0, K * (D // 16), onerow, 0)
                for ch in range(CH):
                    pltpu.sync_copy(dstc_hbm.at[s, ch], dstv)

                    def cstep(b, _):
                        pltpu.sync_copy(rows0, acc.at[dstv.at[b]], add=True)
                        return 0
                    lax.fori_loop(0, CB, cstep, 0)
        plsc.subcore_barrier()

        for r, n in _chunks(RPT, K):
            pltpu.sync_copy(acc.at[pl.ds(r0 + r, n)], rows0.at[pl.ds(0, n)])
            pltpu.sync_copy(rows0.at[pl.ds(0, n)],
                            out_hbm.at[c, pl.ds(r0 + r, n)])

    return pl.kernel(body, out_type=out_type, mesh=mesh,
                     scratch_types=scratch)


_ROW_SPEC = pl.BlockSpec((1000, D), lambda i: (i, 0))
_FULL_SPEC = pl.BlockSpec((D, D), lambda i: (0, 0))
_BIAS_SPEC = pl.BlockSpec((1, D), lambda i: (0, 0))


def _tc_right(h, wr, b):
    def body(h_ref, wr_ref, b_ref, o_ref):
        o_ref[...] = jnp.dot(h_ref[...], wr_ref[...],
                             preferred_element_type=jnp.float32) + b_ref[...]
    return pl.pallas_call(
        body,
        grid=(N // 1000,),
        in_specs=[_ROW_SPEC, _FULL_SPEC, _BIAS_SPEC],
        out_specs=_ROW_SPEC,
        out_shape=jax.ShapeDtypeStruct((N, D), jnp.float32),
    )(h, wr, b)


def _tc_left(a0, cnt, wl, ha, relu):
    def body(a0_ref, c_ref, wl_ref, ha_ref, o_ref):
        inv = 1.0 / jnp.maximum(c_ref[:, 0:1], 1.0)
        agg = a0_ref[...] * inv
        y = jnp.dot(agg, wl_ref[...], preferred_element_type=jnp.float32)
        y = y + ha_ref[...]
        if relu:
            y = jnp.maximum(y, 0.0)
        o_ref[...] = y

    return pl.pallas_call(
        body,
        grid=(N // 1000,),
        in_specs=[_ROW_SPEC, _ROW_SPEC, _FULL_SPEC, _ROW_SPEC],
        out_specs=_ROW_SPEC,
        out_shape=jax.ShapeDtypeStruct((N, D), jnp.float32),
    )(a0, cnt, wl, ha)


def _pad_half(arr, fill, Qc, Qmax):
    cap = NS * Qc * G * K
    pad = cap - arr.shape[0]
    a = jnp.concatenate([arr, jnp.full((pad,), fill, jnp.int32)])
    a = a.reshape(NS, Qc, G, K)
    tail = jnp.full((NS, Qmax + 1 - Qc, G, K), fill, jnp.int32)
    return jnp.concatenate([a, tail], axis=1)


def kernel(x, edge_index, W1l, b1l, W1r, b1r, W2l, b2l, W2r, b2r,
           W3l, b3l, W3r, b3r):
    E = edge_index.shape[1]
    SB = NS * G * K
    src = edge_index[0]
    dst = edge_index[1]

    Q0 = -(-E // SB)
    Q0 += Q0 % 2
    E0 = E
    Q1 = 0
    Qmax = Q0
    src_p = jnp.concatenate([_pad_half(src[:E0], 0, Q0, Qmax),
                             _pad_half(src[E0:], 0, Q1, Qmax)], axis=0)
    dst_p = jnp.concatenate([_pad_half(dst[:E0], N, Q0, Qmax),
                             _pad_half(dst[E0:], N, Q1, Qmax)], axis=0)

    Bc = -(-E // (NS * K))
    CH = -(-Bc // 96)
    CB = -(-Bc // CH)
    EPc = NS * CH * CB * K
    dst_c = jnp.concatenate([dst, jnp.full((EPc - E,), N, jnp.int32)])
    dst_c = dst_c.reshape(NS, CH, CB, K)

    agg_cnt_fn = _make_agg(Q0, Q1, CH, CB)
    agg_fn = _make_agg(Q0, Q1)

    h = x
    cnt = None
    for wl, bl, wr, br, relu in ((W1l, b1l, W1r, b1r, True),
                                 (W2l, b2l, W2r, b2r, True),
                                 (W3l, b3l, W3r, b3r, False)):
        ha = _tc_right(h, wr, (bl + br).reshape(1, D))
        if cnt is None:
            parts = agg_cnt_fn(h, src_p, dst_p, dst_c)
            cnt = parts[1, :N]
        else:
            parts = agg_fn(h, src_p, dst_p)
        h = _tc_left(parts[0, :N], cnt, wl, ha, relu)
    return h
